# baseline (speedup 1.0000x reference)
import functools

import jax
import jax.numpy as jnp
from jax import lax
from jax.experimental import pallas as pl
from jax.experimental.pallas import tpu as pltpu
from jax.experimental.pallas import tpu_sc as plsc

NC = 2
NS = 16
L = 16
NW = NC * NS
B = 16384
D = 32
BPW = B // NW
NG = BPW // L

_mesh = plsc.VectorSubcoreMesh(core_axis_name="c", subcore_axis_name="s")


@functools.partial(
    pl.kernel,
    out_type=jax.ShapeDtypeStruct((B,), jnp.float32),
    mesh=_mesh,
    compiler_params=pltpu.CompilerParams(
        needs_layout_passes=False, use_tc_tiling_on_sc=False),
    scratch_types=[
        pltpu.VMEM((BPW,), jnp.int32),
        pltpu.VMEM((BPW,), jnp.int32),
        pltpu.VMEM((D, BPW), jnp.float32),
        pltpu.VMEM((D, BPW), jnp.float32),
        pltpu.VMEM((D, L), jnp.float32),
        pltpu.VMEM((L,), jnp.float32),
        pltpu.VMEM((BPW,), jnp.float32),
        pltpu.SemaphoreType.DMA,
        pltpu.SemaphoreType.DMA,
    ],
)
def _gmf_sc(uid_hbm, mid_hbm, utab_hbm, mtab_hbm, wb_hbm, bb_hbm,
            out_hbm, uidx, midx, urows, mrows, wv, bv, outv, su, sm):
    wid = lax.axis_index("s") * NC + lax.axis_index("c")
    base = wid * BPW

    pltpu.sync_copy(uid_hbm.at[wid], uidx)
    pltpu.sync_copy(mid_hbm.at[wid], midx)
    pltpu.sync_copy(wb_hbm, wv)
    pltpu.sync_copy(bb_hbm, bv)

    ucopies = [
        pltpu.async_copy(utab_hbm.at[d].at[uidx], urows.at[d], su)
        for d in range(D)
    ]
    mcopies = [
        pltpu.async_copy(mtab_hbm.at[d].at[midx], mrows.at[d], sm)
        for d in range(D)
    ]
    for c in ucopies:
        c.wait()
    for c in mcopies:
        c.wait()

    bias = bv[...]

    def body(g, carry):
        off = g * L
        acc = bias
        for d in range(D):
            acc = acc + urows[d, pl.ds(off, L)] * mrows[d, pl.ds(off, L)] * wv[d, :]
        outv[pl.ds(off, L)] = acc
        return carry

    lax.fori_loop(0, NG, body, 0)
    pltpu.sync_copy(outv, out_hbm.at[pl.ds(base, BPW)])


def kernel(user_ids, movie_ids, user_table, movie_table, W, b):
    uid = user_ids.astype(jnp.int32).reshape(NW, BPW)
    mid = movie_ids.astype(jnp.int32).reshape(NW, BPW)
    w_bcast = jnp.broadcast_to(W.reshape(D, 1), (D, L))
    b_bcast = jnp.broadcast_to(b.reshape(1), (L,))
    return _gmf_sc(uid, mid, user_table.T, movie_table.T, w_bcast, b_bcast)


# speedup vs baseline: 4.7085x; 4.7085x over previous
import functools

import jax
import jax.numpy as jnp
from jax import lax
from jax.experimental import pallas as pl
from jax.experimental.pallas import tpu as pltpu
from jax.experimental.pallas import tpu_sc as plsc

NC = 2
NS = 16
L = 16
NW = NC * NS
B = 16384
D = 32
BPW = B // NW
CHUNK = 128
NIDX = BPW // CHUNK
UNROLL = 4
NITER = BPW // (L * UNROLL)

_mesh = plsc.VectorSubcoreMesh(core_axis_name="c", subcore_axis_name="s")


@functools.partial(
    pl.kernel,
    out_type=jax.ShapeDtypeStruct((B,), jnp.float32),
    mesh=_mesh,
    compiler_params=pltpu.CompilerParams(
        needs_layout_passes=False, use_tc_tiling_on_sc=False),
    scratch_types=[
        pltpu.VMEM((NIDX, CHUNK), jnp.int32),
        pltpu.VMEM((NIDX, CHUNK), jnp.int32),
        pltpu.VMEM((BPW, D), jnp.float32),
        pltpu.VMEM((BPW, D), jnp.float32),
        pltpu.VMEM((D, L), jnp.float32),
        pltpu.VMEM((L,), jnp.float32),
        pltpu.VMEM((BPW,), jnp.float32),
        pltpu.SemaphoreType.DMA,
        pltpu.SemaphoreType.DMA,
    ],
)
def _gmf_sc(uid_hbm, mid_hbm, utab_hbm, mtab_hbm, wb_hbm, bb_hbm,
            out_hbm, uidx, midx, urows, mrows, wv, bv, outv, su, sm):
    wid = lax.axis_index("s") * NC + lax.axis_index("c")
    base = wid * BPW

    pltpu.sync_copy(uid_hbm.at[wid], uidx)
    pltpu.sync_copy(mid_hbm.at[wid], midx)
    ucopies = [
        pltpu.async_copy(utab_hbm.at[uidx.at[j]],
                         urows.at[pl.ds(j * CHUNK, CHUNK)], su)
        for j in range(NIDX)
    ]
    mcopies = [
        pltpu.async_copy(mtab_hbm.at[midx.at[j]],
                         mrows.at[pl.ds(j * CHUNK, CHUNK)], sm)
        for j in range(NIDX)
    ]
    pltpu.sync_copy(wb_hbm, wv)
    pltpu.sync_copy(bb_hbm, bv)
    for c in ucopies:
        c.wait()
    for c in mcopies:
        c.wait()

    bias = bv[...]
    lane = lax.iota(jnp.int32, L)

    def body(it, carry):
        base_row = it * (L * UNROLL)
        rows = [lane + (base_row + u * L) for u in range(UNROLL)]
        accs = [bias] * UNROLL
        for d in range(D):
            wd = wv[d, :]
            col = jnp.full((L,), d, dtype=jnp.int32)
            for u in range(UNROLL):
                uc = plsc.load_gather(urows, [rows[u], col])
                mc = plsc.load_gather(mrows, [rows[u], col])
                accs[u] = accs[u] + uc * mc * wd
        for u in range(UNROLL):
            outv[pl.ds(base_row + u * L, L)] = accs[u]
        return carry

    lax.fori_loop(0, NITER, body, 0)
    pltpu.sync_copy(outv, out_hbm.at[pl.ds(base, BPW)])


def kernel(user_ids, movie_ids, user_table, movie_table, W, b):
    uid = user_ids.astype(jnp.int32).reshape(NW, NIDX, CHUNK)
    mid = movie_ids.astype(jnp.int32).reshape(NW, NIDX, CHUNK)
    w_bcast = jnp.broadcast_to(W.reshape(D, 1), (D, L))
    b_bcast = jnp.broadcast_to(b.reshape(1), (L,))
    return _gmf_sc(uid, mid, user_table, movie_table, w_bcast, b_bcast)
